# baseline (speedup 1.0000x reference)
import functools
import math

import jax
import jax.numpy as jnp
from jax import lax
from jax.experimental import pallas as pl
from jax.experimental.layout import Layout
from jax.experimental.layout import with_layout_constraint
from jax.experimental.pallas import tpu as pltpu
from jax.experimental.pallas import tpu_sc as plsc

D_MODEL = 64
SCALE = math.sqrt(D_MODEL)
B_SZ, S_SZ = 4096, 200
NC, NS, LANES = 2, 16, 16
NW = NC * NS
BPW = B_SZ // NW
NBUF = 4
NGRP = S_SZ // NBUF

_mesh = plsc.VectorSubcoreMesh(core_axis_name="c", subcore_axis_name="s")


@functools.partial(
    pl.kernel,
    mesh=_mesh,
    compiler_params=pltpu.CompilerParams(
        use_tc_tiling_on_sc=False, needs_layout_passes=False),
    out_type=jax.ShapeDtypeStruct((S_SZ, 8, NW, 8, BPW), jnp.float32),
    scratch_types=[
        pltpu.VMEM((S_SZ, BPW), jnp.int32),
        pltpu.VMEM((NBUF, BPW, D_MODEL), jnp.float32),
        pltpu.VMEM((NBUF, 8, 8, BPW), jnp.float32),
        pltpu.SemaphoreType.DMA((NBUF,)),
        pltpu.SemaphoreType.DMA((NBUF,)),
    ],
)
def _emb_lookup(idx_hbm, lut_hbm, out_hbm, idx_v, gbuf, tbuf, gsem, ssem):
    wid = lax.axis_index("s") * NC + lax.axis_index("c")
    wb = wid * BPW
    pltpu.sync_copy(idx_hbm.at[:, pl.ds(wb, BPW)], idx_v)

    lane = lax.iota(jnp.int32, LANES)

    def start_gather(s, b):
        pltpu.make_async_copy(
            lut_hbm.at[idx_v.at[s]], gbuf.at[b], gsem.at[b]).start()

    def wait_gather(s, b):
        pltpu.make_async_copy(
            lut_hbm.at[idx_v.at[s]], gbuf.at[b], gsem.at[b]).wait()

    def out_slot(s):
        return out_hbm.at[s, :, wid]

    def transpose_scale(b):
        def dh_body(dh, carry):
            dvec0 = jnp.full((LANES,), dh * 8, jnp.int32)
            for dl in range(8):
                dvec = dvec0 + dl
                for bg in range(BPW // LANES):
                    rows = bg * LANES + lane
                    v = plsc.load_gather(gbuf.at[b], [rows, dvec])
                    tbuf[b, dh, dl, pl.ds(bg * LANES, LANES)] = v * SCALE
            return carry
        lax.fori_loop(0, D_MODEL // 8, dh_body, 0)

    for b in range(NBUF):
        start_gather(b, b)

    def group(g, carry):
        for b in range(NBUF):
            s = g * NBUF + b
            wait_gather(s, b)
            @pl.when(g > 0)
            def _():
                pltpu.make_async_copy(tbuf.at[b], out_slot(s - NBUF), ssem.at[b]).wait()
            transpose_scale(b)
            pltpu.make_async_copy(tbuf.at[b], out_slot(s), ssem.at[b]).start()
            @pl.when(g < NGRP - 1)
            def _():
                start_gather(s + NBUF, b)
        return carry

    lax.fori_loop(0, NGRP, group, 0)

    for b in range(NBUF):
        s = (NGRP - 1) * NBUF + b
        pltpu.make_async_copy(tbuf.at[b], out_slot(s), ssem.at[b]).wait()


def kernel(indices, lut):
    idx_t = jnp.transpose(indices).astype(jnp.int32)
    out5 = _emb_lookup(idx_t, lut)
    return out5.transpose(2, 4, 0, 1, 3).reshape(B_SZ, S_SZ, D_MODEL)


# speedup vs baseline: 1.7352x; 1.7352x over previous
import functools
import math

import jax
import jax.numpy as jnp
from jax import lax
from jax.experimental import pallas as pl
from jax.experimental.layout import Layout
from jax.experimental.layout import with_layout_constraint
from jax.experimental.pallas import tpu as pltpu
from jax.experimental.pallas import tpu_sc as plsc

D_MODEL = 64
SCALE = math.sqrt(D_MODEL)
B_SZ, S_SZ = 4096, 200
NC, NS, LANES = 2, 16, 16
NW = NC * NS
BPW = B_SZ // NW
NBUF = 4
NGRP = S_SZ // NBUF

_mesh = plsc.VectorSubcoreMesh(core_axis_name="c", subcore_axis_name="s")


@functools.partial(
    pl.kernel,
    mesh=_mesh,
    compiler_params=pltpu.CompilerParams(
        use_tc_tiling_on_sc=False, needs_layout_passes=False),
    out_type=jax.ShapeDtypeStruct((S_SZ, 8, NW, 8, BPW), jnp.float32),
    scratch_types=[
        pltpu.VMEM((S_SZ, BPW), jnp.int32),
        pltpu.VMEM((NBUF, BPW, D_MODEL), jnp.float32),
        pltpu.VMEM((NBUF, 8, 8, BPW + 1), jnp.float32),
        pltpu.SemaphoreType.DMA((NBUF,)),
        pltpu.SemaphoreType.DMA((NBUF,)),
    ],
)
def _emb_lookup(idx_hbm, lut_hbm, out_hbm, idx_v, gbuf, tbuf, gsem, ssem):
    wid = lax.axis_index("s") * NC + lax.axis_index("c")
    wb = wid * BPW
    pltpu.sync_copy(idx_hbm.at[:, pl.ds(wb, BPW)], idx_v)

    lane = lax.iota(jnp.int32, LANES)

    def start_gather(s, b):
        pltpu.make_async_copy(
            lut_hbm.at[idx_v.at[s]], gbuf.at[b], gsem.at[b]).start()

    def wait_gather(s, b):
        pltpu.make_async_copy(
            lut_hbm.at[idx_v.at[s]], gbuf.at[b], gsem.at[b]).wait()

    def out_slot(s):
        return out_hbm.at[s, :, wid]

    def tbuf_src(b):
        return tbuf.at[b, :, :, pl.ds(0, BPW)]

    DH = [(16 * k + lane) // 8 for k in range(D_MODEL // LANES)]
    DL = [lax.rem(16 * k + lane, 8) for k in range(D_MODEL // LANES)]

    def transpose_scale(b):
        def row_body(j, carry):
            jvec = jnp.full((LANES,), j, jnp.int32)
            for k in range(D_MODEL // LANES):
                v = gbuf[b, j, pl.ds(k * LANES, LANES)]
                plsc.store_scatter(tbuf.at[b], [DH[k], DL[k], jvec], v * SCALE)
            return carry
        lax.fori_loop(0, BPW, row_body, 0)

    for b in range(NBUF):
        start_gather(b, b)

    def group(g, carry):
        for b in range(NBUF):
            s = g * NBUF + b
            wait_gather(s, b)
            @pl.when(g > 0)
            def _():
                pltpu.make_async_copy(tbuf_src(b), out_slot(s - NBUF), ssem.at[b]).wait()
            transpose_scale(b)
            pltpu.make_async_copy(tbuf_src(b), out_slot(s), ssem.at[b]).start()
            @pl.when(g < NGRP - 1)
            def _():
                start_gather(s + NBUF, b)
        return carry

    lax.fori_loop(0, NGRP, group, 0)

    for b in range(NBUF):
        s = (NGRP - 1) * NBUF + b
        pltpu.make_async_copy(tbuf_src(b), out_slot(s), ssem.at[b]).wait()


def kernel(indices, lut):
    idx_t = jnp.transpose(indices).astype(jnp.int32)
    out5 = _emb_lookup(idx_t, lut)
    return out5.transpose(2, 4, 0, 1, 3).reshape(B_SZ, S_SZ, D_MODEL)


# speedup vs baseline: 1.7708x; 1.0205x over previous
import functools
import math

import jax
import jax.numpy as jnp
from jax import lax
from jax.experimental import pallas as pl
from jax.experimental.layout import Layout
from jax.experimental.layout import with_layout_constraint
from jax.experimental.pallas import tpu as pltpu
from jax.experimental.pallas import tpu_sc as plsc

D_MODEL = 64
SCALE = math.sqrt(D_MODEL)
B_SZ, S_SZ = 4096, 200
NC, NS, LANES = 2, 16, 16
NW = NC * NS
BPW = B_SZ // NW
NBUF = 4
NGRP = S_SZ // NBUF

_mesh = plsc.VectorSubcoreMesh(core_axis_name="c", subcore_axis_name="s")


@functools.partial(
    pl.kernel,
    mesh=_mesh,
    compiler_params=pltpu.CompilerParams(
        use_tc_tiling_on_sc=False, needs_layout_passes=False),
    out_type=jax.ShapeDtypeStruct((S_SZ, 8, NW, 8, BPW), jnp.float32),
    scratch_types=[
        pltpu.VMEM((S_SZ, BPW), jnp.int32),
        pltpu.VMEM((NBUF, BPW, D_MODEL), jnp.float32),
        pltpu.VMEM((NBUF, 8, 8, BPW + 1), jnp.float32),
        pltpu.SemaphoreType.DMA((NBUF,)),
        pltpu.SemaphoreType.DMA((NBUF,)),
    ],
)
def _emb_lookup(idx_hbm, lut_hbm, out_hbm, idx_v, gbuf, tbuf, gsem, ssem):
    wid = lax.axis_index("s") * NC + lax.axis_index("c")
    wb = wid * BPW
    pltpu.sync_copy(idx_hbm.at[:, pl.ds(wb, BPW)], idx_v)

    lane = lax.iota(jnp.int32, LANES)

    def start_gather(s, b):
        pltpu.make_async_copy(
            lut_hbm.at[idx_v.at[s]], gbuf.at[b], gsem.at[b]).start()

    def wait_gather(s, b):
        pltpu.make_async_copy(
            lut_hbm.at[idx_v.at[s]], gbuf.at[b], gsem.at[b]).wait()

    def out_slot(s):
        return out_hbm.at[s, :, wid]

    def tbuf_src(b):
        return tbuf.at[b, :, :, pl.ds(0, BPW)]

    DH = [(16 * k + lane) // 8 for k in range(D_MODEL // LANES)]
    DL = [lax.rem(16 * k + lane, 8) for k in range(D_MODEL // LANES)]

    def transpose_scale(b):
        def row_body(jg, carry):
            j0 = jg * 4
            jvec0 = jnp.full((LANES,), j0, jnp.int32)
            for dj in range(4):
                j = j0 + dj
                jvec = jvec0 + dj
                for k in range(D_MODEL // LANES):
                    v = gbuf[b, j, pl.ds(k * LANES, LANES)]
                    plsc.store_scatter(tbuf.at[b], [DH[k], DL[k], jvec], v * SCALE)
            return carry
        lax.fori_loop(0, BPW // 4, row_body, 0)

    for b in range(NBUF):
        start_gather(b, b)

    def group(g, carry):
        for b in range(NBUF):
            s = g * NBUF + b
            wait_gather(s, b)
            @pl.when(g > 0)
            def _():
                pltpu.make_async_copy(tbuf_src(b), out_slot(s - NBUF), ssem.at[b]).wait()
            transpose_scale(b)
            pltpu.make_async_copy(tbuf_src(b), out_slot(s), ssem.at[b]).start()
            @pl.when(g < NGRP - 1)
            def _():
                start_gather(s + NBUF, b)
        return carry

    lax.fori_loop(0, NGRP, group, 0)

    for b in range(NBUF):
        s = (NGRP - 1) * NBUF + b
        pltpu.make_async_copy(tbuf_src(b), out_slot(s), ssem.at[b]).wait()


def kernel(indices, lut):
    idx_t = jnp.transpose(indices).astype(jnp.int32)
    out5 = _emb_lookup(idx_t, lut)
    return out5.transpose(2, 4, 0, 1, 3).reshape(B_SZ, S_SZ, D_MODEL)


# speedup vs baseline: 1.8776x; 1.0603x over previous
import functools
import math

import jax
import jax.numpy as jnp
from jax import lax
from jax.experimental import pallas as pl
from jax.experimental.layout import Layout
from jax.experimental.layout import with_layout_constraint
from jax.experimental.pallas import tpu as pltpu
from jax.experimental.pallas import tpu_sc as plsc

D_MODEL = 64
SCALE = math.sqrt(D_MODEL)
B_SZ, S_SZ = 4096, 200
NC, NS, LANES = 2, 16, 16
NW = NC * NS
BPW = B_SZ // NW
NBUF = 4
NGRP = S_SZ // NBUF

_mesh = plsc.VectorSubcoreMesh(core_axis_name="c", subcore_axis_name="s")


@functools.partial(
    pl.kernel,
    mesh=_mesh,
    compiler_params=pltpu.CompilerParams(
        use_tc_tiling_on_sc=False, needs_layout_passes=False),
    out_type=jax.ShapeDtypeStruct((S_SZ, 8, NW, 8, BPW), jnp.float32),
    scratch_types=[
        pltpu.VMEM((S_SZ, BPW), jnp.int32),
        pltpu.VMEM((NBUF, BPW, 2 * D_MODEL), jnp.float32),
        pltpu.VMEM((NBUF, 8, 8, BPW + 1), jnp.float32),
        pltpu.SemaphoreType.DMA((NBUF,)),
        pltpu.SemaphoreType.DMA((NBUF,)),
    ],
)
def _emb_lookup(idx_hbm, lut_hbm, out_hbm, idx_v, gbuf, tbuf, gsem, ssem):
    wid = lax.axis_index("s") * NC + lax.axis_index("c")
    wb = wid * BPW
    pltpu.sync_copy(idx_hbm.at[:, pl.ds(wb, BPW)], idx_v)

    lane = lax.iota(jnp.int32, LANES)

    def start_gather(s, b):
        pltpu.make_async_copy(
            lut_hbm.at[idx_v.at[s]], gbuf.at[b], gsem.at[b]).start()

    def wait_gather(s, b):
        pltpu.make_async_copy(
            lut_hbm.at[idx_v.at[s]], gbuf.at[b], gsem.at[b]).wait()

    def out_slot(s):
        return out_hbm.at[s, :, wid]

    def tbuf_src(b):
        return tbuf.at[b, :, :, pl.ds(0, BPW)]

    DH = [(16 * k + lane) // 8 for k in range(D_MODEL // LANES)]
    DL = [lax.rem(16 * k + lane, 8) for k in range(D_MODEL // LANES)]

    def transpose_scale(b):
        def row_body(jg, carry):
            j0 = jg * 4
            jvec0 = jnp.full((LANES,), j0, jnp.int32)
            for dj in range(4):
                j = j0 + dj
                jvec = jvec0 + dj
                for k in range(D_MODEL // LANES):
                    v = gbuf[b, j, pl.ds(k * LANES, LANES)]
                    plsc.store_scatter(tbuf.at[b], [DH[k], DL[k], jvec], v * SCALE)
            return carry
        lax.fori_loop(0, BPW // 4, row_body, 0)

    for b in range(NBUF):
        start_gather(b, b)

    def group(g, carry):
        for b in range(NBUF):
            s = g * NBUF + b
            wait_gather(s, b)
            @pl.when(g > 0)
            def _():
                pltpu.make_async_copy(tbuf_src(b), out_slot(s - NBUF), ssem.at[b]).wait()
            transpose_scale(b)
            pltpu.make_async_copy(tbuf_src(b), out_slot(s), ssem.at[b]).start()
            @pl.when(g < NGRP - 1)
            def _():
                start_gather(s + NBUF, b)
        return carry

    lax.fori_loop(0, NGRP, group, 0)

    for b in range(NBUF):
        s = (NGRP - 1) * NBUF + b
        pltpu.make_async_copy(tbuf_src(b), out_slot(s), ssem.at[b]).wait()


def kernel(indices, lut):
    idx_t = jnp.transpose(indices).astype(jnp.int32)
    lut128 = jnp.pad(lut, ((0, 0), (0, D_MODEL)))
    out5 = _emb_lookup(idx_t, lut128)
    return out5.transpose(2, 4, 0, 1, 3).reshape(B_SZ, S_SZ, D_MODEL)
